# baseline (speedup 1.0000x reference)
import jax
import jax.numpy as jnp
from jax import lax
from jax.experimental import pallas as pl
from jax.experimental.pallas import tpu as pltpu
from jax.experimental.pallas import tpu_sc as plsc

N_EMB = 1024
DIM = 64
T_TOK = 16384
BLK = 2048
N_BLK = T_TOK // BLK
DECAY_C = 0.99
COMMIT_C = 0.25
EPS_C = 1e-05

_NC = 2
_NS = 16
_NW = _NC * _NS
_CHT = T_TOK // _NW


def _pass1_body(inp_ref, emb_ref, emaw_ref, emacs_ref,
                dist_ref, enc_ref, idx_ref, w_ref, loss_ref, perp_ref,
                es_scr, cnt_scr, dwt_scr, xsum_scr):
    i = pl.program_id(0)
    x2d = inp_ref[...]
    e = emb_ref[...]

    @pl.when(i == 0)
    def _():
        es_scr[...] = jnp.sum(e * e, axis=1, keepdims=True).T

    flat = x2d.T
    xs = jnp.sum(flat * flat, axis=1, keepdims=True)
    xe = lax.dot_general(x2d, e, (((0,), (1,)), ((), ())))
    dist = xs + es_scr[...] - 2.0 * xe
    dist_ref[...] = dist
    idx = jnp.argmin(dist, axis=1).astype(jnp.int32)
    idx_ref[...] = idx.reshape(1, 1, BLK)
    cols = lax.broadcasted_iota(jnp.int32, (BLK, N_EMB), 1)
    ench = (cols == idx[:, None]).astype(jnp.bfloat16)
    enc_ref[...] = ench.astype(jnp.float32)
    ones8 = jnp.ones((8, BLK), jnp.bfloat16)
    cnt = lax.dot_general(ones8, ench, (((1,), (0,)), ((), ())),
                          preferred_element_type=jnp.float32)[0]
    fh = x2d.astype(jnp.bfloat16)
    fl = (x2d - fh.astype(jnp.float32)).astype(jnp.bfloat16)
    dwt = (lax.dot_general(fh, ench, (((1,), (0,)), ((), ())),
                           preferred_element_type=jnp.float32)
           + lax.dot_general(fl, ench, (((1,), (0,)), ((), ())),
                             preferred_element_type=jnp.float32))

    @pl.when(i == 0)
    def _():
        cnt_scr[...] = cnt[None, :]
        dwt_scr[...] = dwt
        xsum_scr[...] = jnp.sum(xs).reshape(1, 1)

    @pl.when(i > 0)
    def _():
        cnt_scr[...] += cnt[None, :]
        dwt_scr[...] += dwt
        xsum_scr[...] += jnp.sum(xs).reshape(1, 1)

    @pl.when(i == N_BLK - 1)
    def _():
        counts = cnt_scr[0, :]
        cs = emacs_ref[0, :] * DECAY_C + (1.0 - DECAY_C) * counts
        n = jnp.sum(cs)
        csn = (cs + EPS_C) / (n + N_EMB * EPS_C) * n
        dwt_full = dwt_scr[...]
        wt = (emaw_ref[...].T * DECAY_C
              + (1.0 - DECAY_C) * dwt_full) / csn[None, :]
        w_ref[...] = jnp.concatenate(
            [wt.T, jnp.zeros((N_EMB, DIM), jnp.float32)], axis=1)
        p = counts * (1.0 / T_TOK)
        perp_ref[...] = jnp.exp(-jnp.sum(p * jnp.log(p + 1e-10))).reshape(1, 1)
        w2 = jnp.sum(wt * wt, axis=0)
        a = jnp.sum(counts * w2)
        b = jnp.sum(dwt_full * wt)
        sq = a - 2.0 * b + xsum_scr[0, 0]
        loss_ref[...] = (sq * (COMMIT_C / (T_TOK * DIM))).reshape(1, 1)


_GCH = 128


def _sc_gather_body(idx_hbm, w_hbm, qtok_hbm, idx_v, rows_v, sem):
    wid = lax.axis_index("s") * _NC + lax.axis_index("c")
    base = wid * _CHT
    pltpu.sync_copy(idx_hbm.at[pl.ds(base, _CHT)], idx_v)
    copies = [
        pltpu.async_copy(w_hbm.at[idx_v.at[pl.ds(k * _GCH, _GCH)]],
                         rows_v.at[pl.ds(k * _GCH, _GCH), :], sem)
        for k in range(_CHT // _GCH)
    ]
    for c in copies:
        c.wait()
    pltpu.sync_copy(rows_v, qtok_hbm.at[pl.ds(base, _CHT), :])


def _transpose_body(qtok_ref, q_ref):
    q_ref[...] = qtok_ref[...][:, :DIM].T


@jax.jit
def kernel(inputs, embedding_weight, ema_w, ema_cluster_size):
    inp2d = inputs.reshape(DIM, T_TOK)

    dist, enc, idx, w, loss, perp = pl.pallas_call(
        _pass1_body,
        grid=(N_BLK,),
        in_specs=[
            pl.BlockSpec((DIM, BLK), lambda i: (0, i)),
            pl.BlockSpec((N_EMB, DIM), lambda i: (0, 0)),
            pl.BlockSpec((N_EMB, DIM), lambda i: (0, 0)),
            pl.BlockSpec((1, N_EMB), lambda i: (0, 0)),
        ],
        out_specs=[
            pl.BlockSpec((BLK, N_EMB), lambda i: (i, 0)),
            pl.BlockSpec((BLK, N_EMB), lambda i: (i, 0)),
            pl.BlockSpec((1, 1, BLK), lambda i: (i, 0, 0)),
            pl.BlockSpec((N_EMB, 2 * DIM), lambda i: (0, 0)),
            pl.BlockSpec((1, 1), lambda i: (0, 0)),
            pl.BlockSpec((1, 1), lambda i: (0, 0)),
        ],
        out_shape=[
            jax.ShapeDtypeStruct((T_TOK, N_EMB), jnp.float32),
            jax.ShapeDtypeStruct((T_TOK, N_EMB), jnp.float32),
            jax.ShapeDtypeStruct((N_BLK, 1, BLK), jnp.int32),
            jax.ShapeDtypeStruct((N_EMB, 2 * DIM), jnp.float32),
            jax.ShapeDtypeStruct((1, 1), jnp.float32),
            jax.ShapeDtypeStruct((1, 1), jnp.float32),
        ],
        scratch_shapes=[
            pltpu.VMEM((1, N_EMB), jnp.float32),
            pltpu.VMEM((1, N_EMB), jnp.float32),
            pltpu.VMEM((DIM, N_EMB), jnp.float32),
            pltpu.VMEM((1, 1), jnp.float32),
        ],
    )(inp2d, embedding_weight, ema_w, ema_cluster_size.reshape(1, N_EMB))

    sc_gather = pl.kernel(
        _sc_gather_body,
        mesh=plsc.VectorSubcoreMesh(core_axis_name="c", subcore_axis_name="s"),
        out_type=jax.ShapeDtypeStruct((T_TOK, 2 * DIM), jnp.float32),
        scratch_types=[
            pltpu.VMEM((_CHT,), jnp.int32),
            pltpu.VMEM((_CHT, 2 * DIM), jnp.float32),
            pltpu.SemaphoreType.DMA,
        ],
    )
    qtok = sc_gather(idx.reshape(T_TOK), w)

    q2d = pl.pallas_call(
        _transpose_body,
        grid=(N_BLK,),
        in_specs=[pl.BlockSpec((BLK, 2 * DIM), lambda i: (i, 0))],
        out_specs=pl.BlockSpec((DIM, BLK), lambda i: (0, i)),
        out_shape=jax.ShapeDtypeStruct((DIM, T_TOK), jnp.float32),
    )(qtok)

    return (loss[0, 0], q2d.reshape(DIM, 1024, 16), perp[0, 0], enc, dist)


# speedup vs baseline: 1.1083x; 1.1083x over previous
import jax
import jax.numpy as jnp
from jax import lax
from jax.experimental import pallas as pl
from jax.experimental.pallas import tpu as pltpu
from jax.experimental.pallas import tpu_sc as plsc

N_EMB = 1024
DIM = 64
T_TOK = 16384
BLK = 2048
N_BLK = T_TOK // BLK
DECAY_C = 0.99
COMMIT_C = 0.25
EPS_C = 1e-05

_NC = 2
_NS = 16
_NW = _NC * _NS
_CHT = T_TOK // _NW


def _pass1_body(inp_ref, emb_ref, emaw_ref, emacs_ref,
                dist_ref, enc_ref, idx_ref, w_ref, loss_ref, perp_ref,
                es_scr, cnt_scr, dwt_scr, xsum_scr):
    i = pl.program_id(0)
    x2d = inp_ref[...]
    e = emb_ref[...]

    @pl.when(i == 0)
    def _():
        es_scr[...] = jnp.sum(e * e, axis=1, keepdims=True).T

    flat = x2d.T
    xs = jnp.sum(flat * flat, axis=1, keepdims=True)
    xe = lax.dot_general(x2d, e, (((0,), (1,)), ((), ())))
    dist = xs + es_scr[...] - 2.0 * xe
    dist_ref[...] = dist
    idx = jnp.argmin(dist, axis=1).astype(jnp.int32)
    idx_ref[...] = idx.reshape(1, 1, BLK)
    cols = lax.broadcasted_iota(jnp.int32, (BLK, N_EMB), 1)
    ench = (cols == idx[:, None]).astype(jnp.bfloat16)
    enc_ref[...] = ench.astype(jnp.float32)
    ones8 = jnp.ones((8, BLK), jnp.bfloat16)
    cnt = lax.dot_general(ones8, ench, (((1,), (0,)), ((), ())),
                          preferred_element_type=jnp.float32)[0]
    fh = x2d.astype(jnp.bfloat16)
    fl = (x2d - fh.astype(jnp.float32)).astype(jnp.bfloat16)
    dwt = (lax.dot_general(fh, ench, (((1,), (0,)), ((), ())),
                           preferred_element_type=jnp.float32)
           + lax.dot_general(fl, ench, (((1,), (0,)), ((), ())),
                             preferred_element_type=jnp.float32))

    @pl.when(i == 0)
    def _():
        cnt_scr[...] = cnt[None, :]
        dwt_scr[...] = dwt
        xsum_scr[...] = jnp.sum(xs).reshape(1, 1)

    @pl.when(i > 0)
    def _():
        cnt_scr[...] += cnt[None, :]
        dwt_scr[...] += dwt
        xsum_scr[...] += jnp.sum(xs).reshape(1, 1)

    @pl.when(i == N_BLK - 1)
    def _():
        counts = cnt_scr[0, :]
        cs = emacs_ref[0, :] * DECAY_C + (1.0 - DECAY_C) * counts
        n = jnp.sum(cs)
        csn = (cs + EPS_C) / (n + N_EMB * EPS_C) * n
        dwt_full = dwt_scr[...]
        wt = (emaw_ref[...].T * DECAY_C
              + (1.0 - DECAY_C) * dwt_full) / csn[None, :]
        w_ref[...] = jnp.concatenate(
            [wt.T, jnp.zeros((N_EMB, DIM), jnp.float32)], axis=1)
        p = counts * (1.0 / T_TOK)
        perp_ref[...] = jnp.exp(-jnp.sum(p * jnp.log(p + 1e-10))).reshape(1, 1)
        w2 = jnp.sum(wt * wt, axis=0)
        a = jnp.sum(counts * w2)
        b = jnp.sum(dwt_full * wt)
        sq = a - 2.0 * b + xsum_scr[0, 0]
        loss_ref[...] = (sq * (COMMIT_C / (T_TOK * DIM))).reshape(1, 1)


_GCH = 128


def _sc_gather_body(idx_hbm, w_hbm, qtok_hbm, idx_v, rows_v, w_sh, sem):
    wid = lax.axis_index("s") * _NC + lax.axis_index("c")
    base = wid * _CHT
    pltpu.sync_copy(idx_hbm.at[pl.ds(base, _CHT)], idx_v)

    @pl.when(lax.axis_index("s") == 0)
    def _():
        pltpu.sync_copy(w_hbm, w_sh)

    plsc.subcore_barrier()
    copies = [
        pltpu.async_copy(w_sh.at[idx_v.at[pl.ds(k * _GCH, _GCH)]],
                         rows_v.at[pl.ds(k * _GCH, _GCH), :], sem)
        for k in range(_CHT // _GCH)
    ]
    for c in copies:
        c.wait()
    pltpu.sync_copy(rows_v, qtok_hbm.at[pl.ds(base, _CHT), :])


def _transpose_body(qtok_ref, q_ref):
    q_ref[...] = qtok_ref[...][:, :DIM].T


@jax.jit
def kernel(inputs, embedding_weight, ema_w, ema_cluster_size):
    inp2d = inputs.reshape(DIM, T_TOK)

    dist, enc, idx, w, loss, perp = pl.pallas_call(
        _pass1_body,
        grid=(N_BLK,),
        in_specs=[
            pl.BlockSpec((DIM, BLK), lambda i: (0, i)),
            pl.BlockSpec((N_EMB, DIM), lambda i: (0, 0)),
            pl.BlockSpec((N_EMB, DIM), lambda i: (0, 0)),
            pl.BlockSpec((1, N_EMB), lambda i: (0, 0)),
        ],
        out_specs=[
            pl.BlockSpec((BLK, N_EMB), lambda i: (i, 0)),
            pl.BlockSpec((BLK, N_EMB), lambda i: (i, 0)),
            pl.BlockSpec((1, 1, BLK), lambda i: (i, 0, 0)),
            pl.BlockSpec((N_EMB, 2 * DIM), lambda i: (0, 0)),
            pl.BlockSpec((1, 1), lambda i: (0, 0)),
            pl.BlockSpec((1, 1), lambda i: (0, 0)),
        ],
        out_shape=[
            jax.ShapeDtypeStruct((T_TOK, N_EMB), jnp.float32),
            jax.ShapeDtypeStruct((T_TOK, N_EMB), jnp.float32),
            jax.ShapeDtypeStruct((N_BLK, 1, BLK), jnp.int32),
            jax.ShapeDtypeStruct((N_EMB, 2 * DIM), jnp.float32),
            jax.ShapeDtypeStruct((1, 1), jnp.float32),
            jax.ShapeDtypeStruct((1, 1), jnp.float32),
        ],
        scratch_shapes=[
            pltpu.VMEM((1, N_EMB), jnp.float32),
            pltpu.VMEM((1, N_EMB), jnp.float32),
            pltpu.VMEM((DIM, N_EMB), jnp.float32),
            pltpu.VMEM((1, 1), jnp.float32),
        ],
    )(inp2d, embedding_weight, ema_w, ema_cluster_size.reshape(1, N_EMB))

    sc_gather = pl.kernel(
        _sc_gather_body,
        mesh=plsc.VectorSubcoreMesh(core_axis_name="c", subcore_axis_name="s"),
        out_type=jax.ShapeDtypeStruct((T_TOK, 2 * DIM), jnp.float32),
        scratch_types=[
            pltpu.VMEM((_CHT,), jnp.int32),
            pltpu.VMEM((_CHT, 2 * DIM), jnp.float32),
            pltpu.VMEM_SHARED((N_EMB, 2 * DIM), jnp.float32),
            pltpu.SemaphoreType.DMA,
        ],
    )
    qtok = sc_gather(idx.reshape(T_TOK), w)

    q2d = pl.pallas_call(
        _transpose_body,
        grid=(N_BLK,),
        in_specs=[pl.BlockSpec((BLK, 2 * DIM), lambda i: (i, 0))],
        out_specs=pl.BlockSpec((DIM, BLK), lambda i: (0, i)),
        out_shape=jax.ShapeDtypeStruct((DIM, T_TOK), jnp.float32),
    )(qtok)

    return (loss[0, 0], q2d.reshape(DIM, 1024, 16), perp[0, 0], enc, dist)


# speedup vs baseline: 1.1275x; 1.0174x over previous
import jax
import jax.numpy as jnp
from jax import lax
from jax.experimental import pallas as pl
from jax.experimental.pallas import tpu as pltpu
from jax.experimental.pallas import tpu_sc as plsc

N_EMB = 1024
DIM = 64
T_TOK = 16384
BLK = 2048
N_BLK = T_TOK // BLK
DECAY_C = 0.99
COMMIT_C = 0.25
EPS_C = 1e-05

_NC = 2
_NS = 16
_NW = _NC * _NS
_CHT = T_TOK // _NW


def _pass1_body(inp_ref, emb_ref, emaw_ref, emacs_ref,
                dist_ref, enc_ref, idx_ref, w_ref, loss_ref, perp_ref,
                es_scr, cnt_scr, dwt_scr, xsum_scr):
    i = pl.program_id(0)
    x2d = inp_ref[...]
    e = emb_ref[...]

    @pl.when(i == 0)
    def _():
        es_scr[...] = jnp.sum(e * e, axis=1, keepdims=True).T

    flat = x2d.T
    xs = jnp.sum(flat * flat, axis=1, keepdims=True)
    xe = lax.dot_general(x2d, e, (((0,), (1,)), ((), ())))
    dist = xs + es_scr[...] - 2.0 * xe
    dist_ref[...] = dist
    idx = jnp.argmin(dist, axis=1).astype(jnp.int32)
    idx_ref[...] = idx.reshape(1, 1, BLK)
    cols = lax.broadcasted_iota(jnp.int32, (BLK, N_EMB), 1)
    ench = (cols == idx[:, None]).astype(jnp.bfloat16)
    enc_ref[...] = ench.astype(jnp.float32)
    ones8 = jnp.ones((8, BLK), jnp.bfloat16)
    cnt = lax.dot_general(ones8, ench, (((1,), (0,)), ((), ())),
                          preferred_element_type=jnp.float32)[0]
    fh = x2d.astype(jnp.bfloat16)
    fl = (x2d - fh.astype(jnp.float32)).astype(jnp.bfloat16)
    dwt = (lax.dot_general(fh, ench, (((1,), (0,)), ((), ())),
                           preferred_element_type=jnp.float32)
           + lax.dot_general(fl, ench, (((1,), (0,)), ((), ())),
                             preferred_element_type=jnp.float32))

    @pl.when(i == 0)
    def _():
        cnt_scr[...] = cnt[None, :]
        dwt_scr[...] = dwt
        xsum_scr[...] = jnp.sum(xs).reshape(1, 1)

    @pl.when(i > 0)
    def _():
        cnt_scr[...] += cnt[None, :]
        dwt_scr[...] += dwt
        xsum_scr[...] += jnp.sum(xs).reshape(1, 1)

    @pl.when(i == N_BLK - 1)
    def _():
        counts = cnt_scr[0, :]
        cs = emacs_ref[0, :] * DECAY_C + (1.0 - DECAY_C) * counts
        n = jnp.sum(cs)
        csn = (cs + EPS_C) / (n + N_EMB * EPS_C) * n
        dwt_full = dwt_scr[...]
        wt = (emaw_ref[...].T * DECAY_C
              + (1.0 - DECAY_C) * dwt_full) / csn[None, :]
        w_ref[...] = jnp.concatenate(
            [wt.T, jnp.zeros((N_EMB, DIM), jnp.float32)], axis=1)
        p = counts * (1.0 / T_TOK)
        perp_ref[...] = jnp.exp(-jnp.sum(p * jnp.log(p + 1e-10))).reshape(1, 1)
        w2 = jnp.sum(wt * wt, axis=0)
        a = jnp.sum(counts * w2)
        b = jnp.sum(dwt_full * wt)
        sq = a - 2.0 * b + xsum_scr[0, 0]
        loss_ref[...] = (sq * (COMMIT_C / (T_TOK * DIM))).reshape(1, 1)


_GCH = 128


def _sc_gather_body(idx_hbm, w_hbm, qtok_hbm, idx_v, rows_v, w_sh, sem):
    wid = lax.axis_index("s") * _NC + lax.axis_index("c")
    base = wid * _CHT
    pltpu.sync_copy(idx_hbm.at[pl.ds(base, _CHT)], idx_v)

    @pl.when(lax.axis_index("s") == 0)
    def _():
        pltpu.sync_copy(w_hbm, w_sh)

    plsc.subcore_barrier()
    copies = [
        pltpu.async_copy(w_sh.at[idx_v.at[pl.ds(k * _GCH, _GCH)]],
                         rows_v.at[pl.ds(k * _GCH, _GCH), :], sem)
        for k in range(_CHT // _GCH)
    ]
    for c in copies:
        c.wait()
    pltpu.sync_copy(rows_v, qtok_hbm.at[pl.ds(base, _CHT), :])


def _transpose_body(qtok_ref, q_ref):
    q_ref[...] = qtok_ref[...][:, :DIM].T


@jax.jit
def kernel(inputs, embedding_weight, ema_w, ema_cluster_size):
    inp2d = inputs.reshape(DIM, T_TOK)

    dist, enc, idx, w, loss, perp = pl.pallas_call(
        _pass1_body,
        grid=(N_BLK,),
        in_specs=[
            pl.BlockSpec((DIM, BLK), lambda i: (0, i)),
            pl.BlockSpec((N_EMB, DIM), lambda i: (0, 0)),
            pl.BlockSpec((N_EMB, DIM), lambda i: (0, 0)),
            pl.BlockSpec((1, N_EMB), lambda i: (0, 0)),
        ],
        out_specs=[
            pl.BlockSpec((BLK, N_EMB), lambda i: (i, 0)),
            pl.BlockSpec((BLK, N_EMB), lambda i: (i, 0)),
            pl.BlockSpec((1, 1, BLK), lambda i: (i, 0, 0)),
            pl.BlockSpec((N_EMB, 2 * DIM), lambda i: (0, 0)),
            pl.BlockSpec((1, 1), lambda i: (0, 0)),
            pl.BlockSpec((1, 1), lambda i: (0, 0)),
        ],
        out_shape=[
            jax.ShapeDtypeStruct((T_TOK, N_EMB), jnp.float32),
            jax.ShapeDtypeStruct((T_TOK, N_EMB), jnp.float32),
            jax.ShapeDtypeStruct((N_BLK, 1, BLK), jnp.int32),
            jax.ShapeDtypeStruct((N_EMB, 2 * DIM), jnp.float32),
            jax.ShapeDtypeStruct((1, 1), jnp.float32),
            jax.ShapeDtypeStruct((1, 1), jnp.float32),
        ],
        scratch_shapes=[
            pltpu.VMEM((1, N_EMB), jnp.float32),
            pltpu.VMEM((1, N_EMB), jnp.float32),
            pltpu.VMEM((DIM, N_EMB), jnp.float32),
            pltpu.VMEM((1, 1), jnp.float32),
        ],
    )(inp2d, embedding_weight, ema_w, ema_cluster_size.reshape(1, N_EMB))

    sc_gather = pl.kernel(
        _sc_gather_body,
        mesh=plsc.VectorSubcoreMesh(core_axis_name="c", subcore_axis_name="s"),
        out_type=jax.ShapeDtypeStruct((T_TOK, 2 * DIM), jnp.float32),
        scratch_types=[
            pltpu.VMEM((_CHT,), jnp.int32),
            pltpu.VMEM((_CHT, 2 * DIM), jnp.float32),
            pltpu.VMEM_SHARED((N_EMB, 2 * DIM), jnp.float32),
            pltpu.SemaphoreType.DMA,
        ],
    )
    qtok = sc_gather(idx.reshape(T_TOK), w)

    q2d = pl.pallas_call(
        _transpose_body,
        grid=(4,),
        in_specs=[pl.BlockSpec((T_TOK // 4, 2 * DIM), lambda i: (i, 0))],
        out_specs=pl.BlockSpec((DIM, T_TOK // 4), lambda i: (0, i)),
        out_shape=jax.ShapeDtypeStruct((DIM, T_TOK), jnp.float32),
    )(qtok)

    return (loss[0, 0], q2d.reshape(DIM, 1024, 16), perp[0, 0], enc, dist)


# speedup vs baseline: 1.1364x; 1.0079x over previous
import jax
import jax.numpy as jnp
from jax import lax
from jax.experimental import pallas as pl
from jax.experimental.pallas import tpu as pltpu
from jax.experimental.pallas import tpu_sc as plsc

N_EMB = 1024
DIM = 64
T_TOK = 16384
BLK = 2048
N_BLK = T_TOK // BLK
DECAY_C = 0.99
COMMIT_C = 0.25
EPS_C = 1e-05

_NC = 2
_NS = 16
_NW = _NC * _NS
_CHT = T_TOK // _NW


def _pass1_body(inp_ref, emb_ref, emaw_ref, emacs_ref,
                dist_ref, enc_ref, idx_ref, w_ref, loss_ref, perp_ref,
                es_scr, cnt_scr, dwt_scr, xsum_scr):
    i = pl.program_id(0)
    x2d = inp_ref[...]
    e = emb_ref[...]

    @pl.when(i == 0)
    def _():
        es_scr[...] = jnp.sum(e * e, axis=1, keepdims=True).T

    flat = x2d.T
    xs = jnp.sum(flat * flat, axis=1, keepdims=True)
    xe = lax.dot_general(x2d, e, (((0,), (1,)), ((), ())))
    dist = xs + es_scr[...] - 2.0 * xe
    dist_ref[...] = dist
    idx = jnp.argmin(dist, axis=1).astype(jnp.int32)
    idx_ref[...] = idx.reshape(1, 1, BLK)
    cols = lax.broadcasted_iota(jnp.int32, (BLK, N_EMB), 1)
    ench = (cols == idx[:, None]).astype(jnp.bfloat16)
    enc_ref[...] = ench.astype(jnp.float32)
    ones8 = jnp.ones((8, BLK), jnp.bfloat16)
    cnt = lax.dot_general(ones8, ench, (((1,), (0,)), ((), ())),
                          preferred_element_type=jnp.float32)[0]
    fh = x2d.astype(jnp.bfloat16)
    fl = (x2d - fh.astype(jnp.float32)).astype(jnp.bfloat16)
    dwt = (lax.dot_general(fh, ench, (((1,), (0,)), ((), ())),
                           preferred_element_type=jnp.float32)
           + lax.dot_general(fl, ench, (((1,), (0,)), ((), ())),
                             preferred_element_type=jnp.float32))

    @pl.when(i == 0)
    def _():
        cnt_scr[...] = cnt[None, :]
        dwt_scr[...] = dwt
        xsum_scr[...] = jnp.sum(xs).reshape(1, 1)

    @pl.when(i > 0)
    def _():
        cnt_scr[...] += cnt[None, :]
        dwt_scr[...] += dwt
        xsum_scr[...] += jnp.sum(xs).reshape(1, 1)

    @pl.when(i == N_BLK - 1)
    def _():
        counts = cnt_scr[0, :]
        cs = emacs_ref[0, :] * DECAY_C + (1.0 - DECAY_C) * counts
        n = jnp.sum(cs)
        csn = (cs + EPS_C) / (n + N_EMB * EPS_C) * n
        dwt_full = dwt_scr[...]
        wt = (emaw_ref[...].T * DECAY_C
              + (1.0 - DECAY_C) * dwt_full) / csn[None, :]
        w_ref[...] = jnp.concatenate(
            [wt.T, jnp.zeros((N_EMB, DIM), jnp.float32)], axis=1)
        p = counts * (1.0 / T_TOK)
        perp_ref[...] = jnp.exp(-jnp.sum(p * jnp.log(p + 1e-10))).reshape(1, 1)
        w2 = jnp.sum(wt * wt, axis=0)
        a = jnp.sum(counts * w2)
        b = jnp.sum(dwt_full * wt)
        sq = a - 2.0 * b + xsum_scr[0, 0]
        loss_ref[...] = (sq * (COMMIT_C / (T_TOK * DIM))).reshape(1, 1)


_GCH = 128


def _sc_gather_body(idx_hbm, w_hbm, qtok_hbm, idx_v, rows_v, w_sh, sem):
    wid = lax.axis_index("s") * _NC + lax.axis_index("c")
    base = wid * _CHT
    pltpu.sync_copy(idx_hbm.at[pl.ds(base, _CHT)], idx_v)

    @pl.when(lax.axis_index("s") == 0)
    def _():
        pltpu.sync_copy(w_hbm, w_sh)

    plsc.subcore_barrier()
    copies = [
        pltpu.async_copy(w_sh.at[idx_v.at[pl.ds(k * _GCH, _GCH)]],
                         rows_v.at[pl.ds(k * _GCH, _GCH), :], sem)
        for k in range(_CHT // _GCH)
    ]
    for c in copies:
        c.wait()
    pltpu.sync_copy(rows_v, qtok_hbm.at[pl.ds(base, _CHT), :])


def _transpose_body(qtok_ref, q_ref):
    q_ref[...] = qtok_ref[...][:, :DIM].T


@jax.jit
def kernel(inputs, embedding_weight, ema_w, ema_cluster_size):
    inp2d = inputs.reshape(DIM, T_TOK)

    dist, enc, idx, w, loss, perp = pl.pallas_call(
        _pass1_body,
        grid=(N_BLK,),
        in_specs=[
            pl.BlockSpec((DIM, BLK), lambda i: (0, i)),
            pl.BlockSpec((N_EMB, DIM), lambda i: (0, 0)),
            pl.BlockSpec((N_EMB, DIM), lambda i: (0, 0)),
            pl.BlockSpec((1, N_EMB), lambda i: (0, 0)),
        ],
        out_specs=[
            pl.BlockSpec((BLK, N_EMB), lambda i: (i, 0)),
            pl.BlockSpec((BLK, N_EMB), lambda i: (i, 0)),
            pl.BlockSpec((1, 1, BLK), lambda i: (i, 0, 0)),
            pl.BlockSpec((N_EMB, 2 * DIM), lambda i: (0, 0)),
            pl.BlockSpec((1, 1), lambda i: (0, 0)),
            pl.BlockSpec((1, 1), lambda i: (0, 0)),
        ],
        out_shape=[
            jax.ShapeDtypeStruct((T_TOK, N_EMB), jnp.float32),
            jax.ShapeDtypeStruct((T_TOK, N_EMB), jnp.float32),
            jax.ShapeDtypeStruct((N_BLK, 1, BLK), jnp.int32),
            jax.ShapeDtypeStruct((N_EMB, 2 * DIM), jnp.float32),
            jax.ShapeDtypeStruct((1, 1), jnp.float32),
            jax.ShapeDtypeStruct((1, 1), jnp.float32),
        ],
        scratch_shapes=[
            pltpu.VMEM((1, N_EMB), jnp.float32),
            pltpu.VMEM((1, N_EMB), jnp.float32),
            pltpu.VMEM((DIM, N_EMB), jnp.float32),
            pltpu.VMEM((1, 1), jnp.float32),
        ],
    )(inp2d, embedding_weight, ema_w, ema_cluster_size.reshape(1, N_EMB))

    sc_gather = pl.kernel(
        _sc_gather_body,
        mesh=plsc.VectorSubcoreMesh(core_axis_name="c", subcore_axis_name="s"),
        out_type=jax.ShapeDtypeStruct((T_TOK, 2 * DIM), jnp.float32),
        scratch_types=[
            pltpu.VMEM((_CHT,), jnp.int32),
            pltpu.VMEM((_CHT, 2 * DIM), jnp.float32),
            pltpu.VMEM_SHARED((N_EMB, 2 * DIM), jnp.float32),
            pltpu.SemaphoreType.DMA,
        ],
    )
    qtok = sc_gather(idx.reshape(T_TOK), w)

    q2d = pl.pallas_call(
        _transpose_body,
        grid=(2,),
        in_specs=[pl.BlockSpec((T_TOK // 2, 2 * DIM), lambda i: (i, 0))],
        out_specs=pl.BlockSpec((DIM, T_TOK // 2), lambda i: (0, i)),
        out_shape=jax.ShapeDtypeStruct((DIM, T_TOK), jnp.float32),
    )(qtok)

    return (loss[0, 0], q2d.reshape(DIM, 1024, 16), perp[0, 0], enc, dist)
